# baseline (speedup 1.0000x reference)
import jax
import jax.numpy as jnp
from jax.experimental import pallas as pl

_BS = 512
_K = 4


def _fused_kernel(*refs):
    emb_refs = refs[:_K]
    mask_ref, wt_ref, out_ref = refs[_K], refs[_K + 1], refs[_K + 2]
    w = wt_ref[...]
    for k in range(_K):
        x = emb_refs[k][...].astype(jnp.bfloat16)
        mm = jnp.dot(x, w, preferred_element_type=jnp.float32)
        m = mask_ref[k * _BS:(k + 1) * _BS, :] > 0
        out_ref[k * _BS:(k + 1) * _BS, :] = jnp.where(m, mm, -jnp.inf)


def kernel(emb_sentences, att_sentences, W):
    B, S, D = emb_sentences.shape
    L = W.shape[0]
    N = B * S
    emb = emb_sentences.reshape(N, D)
    mask = att_sentences.reshape(N, 1).astype(jnp.float32)
    wt = W.T.astype(jnp.bfloat16)

    rows_per_step = _K * _BS
    grid = (N // rows_per_step,)

    def emb_spec(k):
        return pl.BlockSpec((_BS, D), lambda i, k=k: (_K * i + k, 0))

    out = pl.pallas_call(
        _fused_kernel,
        grid=grid,
        in_specs=[emb_spec(k) for k in range(_K)]
        + [
            pl.BlockSpec((rows_per_step, 1), lambda i: (i, 0)),
            pl.BlockSpec((D, L), lambda i: (0, 0)),
        ],
        out_specs=pl.BlockSpec((rows_per_step, L), lambda i: (i, 0)),
        out_shape=jax.ShapeDtypeStruct((N, L), jnp.float32),
    )(*([emb] * _K), mask, wt)
    return out.reshape(B, S, L)
